# baseline (speedup 1.0000x reference)
import functools

import jax
import jax.numpy as jnp
from jax import lax
from jax.experimental import pallas as pl
from jax.experimental.pallas import tpu as pltpu
from jax.experimental.pallas import tpu_sc as plsc

B, T, D = 4096, 200, 64
R = B * T
NC, NS = 2, 16
NW = NC * NS
BW = B // NW
LANES = 16
NJG = BW // LANES
NPAIR = 1000000 // 2


def _make_kernel():
  mesh = plsc.VectorSubcoreMesh(
      core_axis_name="c", subcore_axis_name="s",
      num_cores=NC, num_subcores=NS)

  @functools.partial(
      pl.kernel,
      out_type=jax.ShapeDtypeStruct((T, D, B), jnp.float32),
      mesh=mesh,
      scratch_types=[
          pltpu.VMEM((T, BW), jnp.int32),
          pltpu.VMEM((T, 2 * D), jnp.float32),
          pltpu.VMEM((BW,), jnp.int32),
          pltpu.VMEM((BW,), jnp.int32),
          pltpu.VMEM((BW, 2 * D), jnp.float32),
          pltpu.VMEM((BW, 2 * D), jnp.float32),
          pltpu.VMEM((D, BW), jnp.float32),
          pltpu.VMEM((D, BW), jnp.float32),
          pltpu.SemaphoreType.DMA,
          pltpu.SemaphoreType.DMA,
          pltpu.SemaphoreType.DMA,
          pltpu.SemaphoreType.DMA,
      ],
      compiler_params=pltpu.CompilerParams(needs_layout_passes=False),
  )
  def k(xT_hbm, emb2_hbm, posP_hbm, out_hbm, x_blk, pos_v,
        idx0, idx1, rows0, rows1, blk0, blk1,
        sem_g0, sem_g1, sem_s0, sem_s1):
    wid = lax.axis_index("s") * NC + lax.axis_index("c")
    b0 = wid * BW
    idxs = (idx0, idx1)
    rows = (rows0, rows1)
    blks = (blk0, blk1)
    sem_g = (sem_g0, sem_g1)
    sem_s = (sem_s0, sem_s1)
    iot = lax.iota(jnp.int32, LANES)

    pltpu.sync_copy(xT_hbm.at[:, pl.ds(b0, BW)], x_blk)
    pltpu.sync_copy(posP_hbm, pos_v)

    def prep_idx(t, slot):
      for jg in range(NJG):
        sl = pl.ds(jg * LANES, LANES)
        idxs[slot][sl] = jnp.right_shift(x_blk[t, sl], 1)

    def issue_gather(slot):
      pltpu.async_copy(emb2_hbm.at[idxs[slot]], rows[slot], sem_g[slot])

    def wait_gather(slot):
      pltpu.make_async_copy(
          emb2_hbm.at[idxs[slot]], rows[slot], sem_g[slot]).wait()

    def issue_store(t, slot):
      pltpu.async_copy(
          blks[slot], out_hbm.at[t, :, pl.ds(b0, BW)], sem_s[slot])

    def wait_store(slot):
      pltpu.make_async_copy(
          blks[slot], out_hbm.at[0, :, pl.ds(b0, BW)], sem_s[slot]).wait()

    def transpose_add(t, slot):
      rv = rows[slot]
      bv = blks[slot]
      tsp = jnp.full((LANES,), t, jnp.int32)

      def d_body(d, carry):
        dsp = jnp.full((LANES,), d, jnp.int32)
        pv = plsc.load_gather(pos_v, [tsp, dsp])
        for jg in range(NJG):
          sl = pl.ds(jg * LANES, LANES)
          par64 = jnp.left_shift(jnp.bitwise_and(x_blk[t, sl], 1), 6)
          jvec = iot + (jg * LANES)
          g = plsc.load_gather(rv, [jvec, par64 + d])
          bv[d, sl] = g + pv
        return carry

      lax.fori_loop(0, D, d_body, 0)

    prep_idx(0, 0)
    issue_gather(0)
    prep_idx(1, 1)
    issue_gather(1)

    def step(i, carry):
      t0 = 2 * i
      t1 = t0 + 1

      wait_gather(0)

      @pl.when(i > 0)
      def _():
        wait_store(0)
      transpose_add(t0, 0)

      @pl.when(i < T // 2 - 1)
      def _():
        prep_idx(t0 + 2, 0)
        issue_gather(0)
      issue_store(t0, 0)

      wait_gather(1)

      @pl.when(i > 0)
      def _():
        wait_store(1)
      transpose_add(t1, 1)

      @pl.when(i < T // 2 - 1)
      def _():
        prep_idx(t1 + 2, 1)
        issue_gather(1)
      issue_store(t1, 1)
      return carry

    lax.fori_loop(0, T // 2, step, 0)
    wait_store(0)
    wait_store(1)

  return k


_kernel = _make_kernel()


@jax.jit
def kernel(x, emb_table, pos_table):
  emb2 = emb_table.reshape(NPAIR, 2 * D)
  xT = x.T
  posP = jnp.pad(pos_table, ((0, 0), (0, D)))
  out3 = _kernel(xT, emb2, posP)
  return out3.transpose(2, 0, 1)


# speedup vs baseline: 1.3731x; 1.3731x over previous
import functools

import jax
import jax.numpy as jnp
from jax import lax
from jax.experimental import pallas as pl
from jax.experimental.pallas import tpu as pltpu
from jax.experimental.pallas import tpu_sc as plsc

B, T, D = 4096, 200, 64
R = B * T
NC, NS = 2, 16
NW = NC * NS
BW = B // NW
LANES = 16
NJG = BW // LANES
NPAIR = 1000000 // 2


def _make_kernel():
  mesh = plsc.VectorSubcoreMesh(
      core_axis_name="c", subcore_axis_name="s",
      num_cores=NC, num_subcores=NS)

  @functools.partial(
      pl.kernel,
      out_type=jax.ShapeDtypeStruct((T, D, B), jnp.float32),
      mesh=mesh,
      scratch_types=[
          pltpu.VMEM((T, BW), jnp.int32),
          pltpu.VMEM((T, 2 * D), jnp.float32),
          pltpu.VMEM((BW,), jnp.int32),
          pltpu.VMEM((BW,), jnp.int32),
          pltpu.VMEM((BW, 2 * D), jnp.float32),
          pltpu.VMEM((BW, 2 * D), jnp.float32),
          pltpu.VMEM((D, BW), jnp.float32),
          pltpu.VMEM((D, BW), jnp.float32),
          pltpu.SemaphoreType.DMA,
          pltpu.SemaphoreType.DMA,
          pltpu.SemaphoreType.DMA,
          pltpu.SemaphoreType.DMA,
      ],
      compiler_params=pltpu.CompilerParams(needs_layout_passes=False),
  )
  def k(xT_hbm, emb2_hbm, posP_hbm, out_hbm, x_blk, pos_v,
        idx0, idx1, rows0, rows1, blk0, blk1,
        sem_g0, sem_g1, sem_s0, sem_s1):
    wid = lax.axis_index("s") * NC + lax.axis_index("c")
    b0 = wid * BW
    idxs = (idx0, idx1)
    rows = (rows0, rows1)
    blks = (blk0, blk1)
    sem_g = (sem_g0, sem_g1)
    sem_s = (sem_s0, sem_s1)
    iot = lax.iota(jnp.int32, LANES)

    pltpu.sync_copy(xT_hbm.at[:, pl.ds(b0, BW)], x_blk)
    pltpu.sync_copy(posP_hbm, pos_v)

    def prep_idx(t, slot):
      for jg in range(NJG):
        sl = pl.ds(jg * LANES, LANES)
        idxs[slot][sl] = jnp.right_shift(x_blk[t, sl], 1)

    def issue_gather(slot):
      pltpu.async_copy(emb2_hbm.at[idxs[slot]], rows[slot], sem_g[slot])

    def wait_gather(slot):
      pltpu.make_async_copy(
          emb2_hbm.at[idxs[slot]], rows[slot], sem_g[slot]).wait()

    def issue_store(t, slot):
      pltpu.async_copy(
          blks[slot], out_hbm.at[t, :, pl.ds(b0, BW)], sem_s[slot])

    def wait_store(slot):
      pltpu.make_async_copy(
          blks[slot], out_hbm.at[0, :, pl.ds(b0, BW)], sem_s[slot]).wait()

    def transpose_add(t, slot):
      rv = rows[slot]
      bv = blks[slot]
      tsp = jnp.full((LANES,), t, jnp.int32)
      jvecs = [iot + (jg * LANES) for jg in range(NJG)]
      pars = []
      for jg in range(NJG):
        sl = pl.ds(jg * LANES, LANES)
        pars.append(jnp.left_shift(jnp.bitwise_and(x_blk[t, sl], 1), 6))

      UN = 4

      def d_body(di, carry):
        for k in range(UN):
          d = di * UN + k
          dsp = jnp.full((LANES,), d, jnp.int32)
          pv = plsc.load_gather(pos_v, [tsp, dsp])
          for jg in range(NJG):
            g = plsc.load_gather(rv, [jvecs[jg], pars[jg] + d])
            bv[d, pl.ds(jg * LANES, LANES)] = g + pv
        return carry

      lax.fori_loop(0, D // UN, d_body, 0)

    prep_idx(0, 0)
    issue_gather(0)
    prep_idx(1, 1)
    issue_gather(1)

    def step(i, carry):
      t0 = 2 * i
      t1 = t0 + 1

      wait_gather(0)

      @pl.when(i > 0)
      def _():
        wait_store(0)
      transpose_add(t0, 0)

      @pl.when(i < T // 2 - 1)
      def _():
        prep_idx(t0 + 2, 0)
        issue_gather(0)
      issue_store(t0, 0)

      wait_gather(1)

      @pl.when(i > 0)
      def _():
        wait_store(1)
      transpose_add(t1, 1)

      @pl.when(i < T // 2 - 1)
      def _():
        prep_idx(t1 + 2, 1)
        issue_gather(1)
      issue_store(t1, 1)
      return carry

    lax.fori_loop(0, T // 2, step, 0)
    wait_store(0)
    wait_store(1)

  return k


_kernel = _make_kernel()


@jax.jit
def kernel(x, emb_table, pos_table):
  emb2 = emb_table.reshape(NPAIR, 2 * D)
  xT = x.T
  posP = jnp.pad(pos_table, ((0, 0), (0, D)))
  out3 = _kernel(xT, emb2, posP)
  return out3.transpose(2, 0, 1)


# speedup vs baseline: 1.5502x; 1.1290x over previous
import functools

import jax
import jax.numpy as jnp
from jax import lax
from jax.experimental import pallas as pl
from jax.experimental.pallas import tpu as pltpu
from jax.experimental.pallas import tpu_sc as plsc

B, T, D = 4096, 200, 64
R = B * T
NC, NS = 2, 16
NW = NC * NS
BW = B // NW
LANES = 16
NDG = D // LANES
NJG = BW // LANES


def _make_kernel():
  mesh = plsc.VectorSubcoreMesh(
      core_axis_name="c", subcore_axis_name="s",
      num_cores=NC, num_subcores=NS)

  @functools.partial(
      pl.kernel,
      out_type=jax.ShapeDtypeStruct((T, D, B), jnp.float32),
      mesh=mesh,
      scratch_types=[
          pltpu.VMEM((T, BW), jnp.int32),
          pltpu.VMEM((T, 2 * D), jnp.float32),
          pltpu.VMEM((BW,), jnp.int32),
          pltpu.VMEM((BW,), jnp.int32),
          pltpu.VMEM((BW, 2 * D), jnp.float32),
          pltpu.VMEM((BW, 2 * D), jnp.float32),
          pltpu.VMEM((D, BW), jnp.float32),
          pltpu.VMEM((D, BW), jnp.float32),
          pltpu.SemaphoreType.DMA,
          pltpu.SemaphoreType.DMA,
          pltpu.SemaphoreType.DMA,
          pltpu.SemaphoreType.DMA,
      ],
      compiler_params=pltpu.CompilerParams(needs_layout_passes=False),
  )
  def k(xT_hbm, emb2, posP_hbm, out_hbm, x_blk, pos_v, idx0, idx1,
        rows0, rows1, blk0, blk1, sem_g0, sem_g1, sem_s0, sem_s1):
    wid = lax.axis_index("s") * NC + lax.axis_index("c")
    b0 = wid * BW
    idxs = (idx0, idx1)
    rows = (rows0, rows1)
    blks = (blk0, blk1)
    sem_g = (sem_g0, sem_g1)
    sem_s = (sem_s0, sem_s1)
    iot = lax.iota(jnp.int32, LANES)
    pltpu.sync_copy(xT_hbm.at[:, pl.ds(b0, BW)], x_blk)
    pltpu.sync_copy(posP_hbm, pos_v)

    def issue_gather(t, slot):
      for jg in range(NJG):
        sl = pl.ds(jg * LANES, LANES)
        idxs[slot][sl] = jnp.right_shift(x_blk[t, sl], 1)
      pltpu.async_copy(emb2.at[idxs[slot]], rows[slot], sem_g[slot])

    def wait_gather(slot):
      pltpu.make_async_copy(
          emb2.at[idxs[slot]], rows[slot], sem_g[slot]).wait()

    def issue_store(t, slot):
      pltpu.async_copy(
          blks[slot], out_hbm.at[t, :, pl.ds(b0, BW)], sem_s[slot])

    def wait_store(slot):
      pltpu.make_async_copy(
          blks[slot], out_hbm.at[0, :, pl.ds(b0, BW)], sem_s[slot]).wait()

    dvecs = [iot + (dg * LANES) for dg in range(NDG)]

    def transpose_add(t, slot):
      rv = rows[slot]
      bv = blks[slot]
      pvs = [pos_v[t, pl.ds(dg * LANES, LANES)] for dg in range(NDG)]

      def jg_body(jg, carry):
        sl = pl.ds(jg * LANES, LANES)
        par_vec = jnp.left_shift(jnp.bitwise_and(x_blk[t, sl], 1), 6)
        for k in range(LANES):
          j = jg * LANES + k
          jsp = jnp.full((LANES,), j, jnp.int32)
          par64 = par_vec[k]
          for dg in range(NDG):
            v = rv[j, pl.ds(par64 + dg * LANES, LANES)] + pvs[dg]
            plsc.store_scatter(bv, [dvecs[dg], jsp], v)
        return carry

      lax.fori_loop(0, NJG, jg_body, 0, unroll=2)

    issue_gather(0, 0)
    issue_gather(1, 1)

    def step(i, carry):
      t0 = 2 * i
      t1 = t0 + 1

      wait_gather(0)

      @pl.when(i > 0)
      def _():
        wait_store(0)
      transpose_add(t0, 0)

      @pl.when(i < T // 2 - 1)
      def _():
        issue_gather(t0 + 2, 0)
      issue_store(t0, 0)

      wait_gather(1)

      @pl.when(i > 0)
      def _():
        wait_store(1)
      transpose_add(t1, 1)

      @pl.when(i < T // 2 - 1)
      def _():
        issue_gather(t1 + 2, 1)
      issue_store(t1, 1)
      return carry

    lax.fori_loop(0, T // 2, step, 0)
    wait_store(0)
    wait_store(1)

  return k


_kernel = _make_kernel()


@jax.jit
def kernel(x, emb_table, pos_table):
  xT = x.T
  posP = jnp.pad(pos_table, ((0, 0), (0, D)))
  emb2 = emb_table.reshape(500000, 2 * D)
  out3 = _kernel(xT, emb2, posP)
  return out3.transpose(2, 0, 1)


# speedup vs baseline: 1.8588x; 1.1990x over previous
import functools

import jax
import jax.numpy as jnp
from jax import lax
from jax.experimental import pallas as pl
from jax.experimental.pallas import tpu as pltpu
from jax.experimental.pallas import tpu_sc as plsc

B, T, D = 4096, 200, 64
R = B * T
NC, NS = 2, 16
NW = NC * NS
BW = B // NW
LANES = 16
NDG = D // LANES
NJG = BW // LANES


def _make_kernel():
  mesh = plsc.VectorSubcoreMesh(
      core_axis_name="c", subcore_axis_name="s",
      num_cores=NC, num_subcores=NS)

  @functools.partial(
      pl.kernel,
      out_type=jax.ShapeDtypeStruct((T, D, B), jnp.float32),
      mesh=mesh,
      scratch_types=[
          pltpu.VMEM((T, BW), jnp.int32),
          pltpu.VMEM((T, 2 * D), jnp.float32),
          pltpu.VMEM((BW,), jnp.int32),
          pltpu.VMEM((BW,), jnp.int32),
          pltpu.VMEM((BW, 2 * D), jnp.float32),
          pltpu.VMEM((BW, 2 * D), jnp.float32),
          pltpu.VMEM((D, BW), jnp.float32),
          pltpu.VMEM((D, BW), jnp.float32),
          pltpu.SemaphoreType.DMA,
          pltpu.SemaphoreType.DMA,
          pltpu.SemaphoreType.DMA,
          pltpu.SemaphoreType.DMA,
      ],
      compiler_params=pltpu.CompilerParams(needs_layout_passes=False),
  )
  def k(xT_hbm, emb2, posP_hbm, out_hbm, x_blk, pos_v, idx0, idx1,
        rows0, rows1, blk0, blk1, sem_g0, sem_g1, sem_s0, sem_s1):
    wid = lax.axis_index("s") * NC + lax.axis_index("c")
    b0 = wid * BW
    idxs = (idx0, idx1)
    rows = (rows0, rows1)
    blks = (blk0, blk1)
    sem_g = (sem_g0, sem_g1)
    sem_s = (sem_s0, sem_s1)
    iot = lax.iota(jnp.int32, LANES)
    pltpu.sync_copy(xT_hbm.at[:, pl.ds(b0, BW)], x_blk)
    pltpu.sync_copy(posP_hbm, pos_v)

    def issue_gather(t, slot):
      for jg in range(NJG):
        sl = pl.ds(jg * LANES, LANES)
        idxs[slot][sl] = jnp.right_shift(x_blk[t, sl], 1)
      pltpu.async_copy(emb2.at[idxs[slot]], rows[slot], sem_g[slot])

    def wait_gather(slot):
      pltpu.make_async_copy(
          emb2.at[idxs[slot]], rows[slot], sem_g[slot]).wait()

    def issue_store(t, slot):
      pltpu.async_copy(
          blks[slot], out_hbm.at[t, :, pl.ds(b0, BW)], sem_s[slot])

    def wait_store(slot):
      pltpu.make_async_copy(
          blks[slot], out_hbm.at[0, :, pl.ds(b0, BW)], sem_s[slot]).wait()

    dvecs = [iot + (dg * LANES) for dg in range(NDG)]

    def transpose_add(t, slot):
      rv = rows[slot]
      bv = blks[slot]
      pvs = [pos_v[t, pl.ds(dg * LANES, LANES)] for dg in range(NDG)]

      @plsc.parallel_loop(0, NJG, unroll=2)
      def _(jg):
        sl = pl.ds(jg * LANES, LANES)
        par_vec = jnp.left_shift(jnp.bitwise_and(x_blk[t, sl], 1), 6)
        for k in range(LANES):
          j = jg * LANES + k
          jsp = jnp.full((LANES,), j, jnp.int32)
          par64 = par_vec[k]
          for dg in range(NDG):
            v = rv[j, pl.ds(par64 + dg * LANES, LANES)] + pvs[dg]
            plsc.store_scatter(bv, [dvecs[dg], jsp], v)

    issue_gather(0, 0)
    issue_gather(1, 1)

    def step(i, carry):
      t0 = 2 * i
      t1 = t0 + 1

      wait_gather(0)

      @pl.when(i > 0)
      def _():
        wait_store(0)
      transpose_add(t0, 0)

      @pl.when(i < T // 2 - 1)
      def _():
        issue_gather(t0 + 2, 0)
      issue_store(t0, 0)

      wait_gather(1)

      @pl.when(i > 0)
      def _():
        wait_store(1)
      transpose_add(t1, 1)

      @pl.when(i < T // 2 - 1)
      def _():
        issue_gather(t1 + 2, 1)
      issue_store(t1, 1)
      return carry

    lax.fori_loop(0, T // 2, step, 0)
    wait_store(0)
    wait_store(1)

  return k


_kernel = _make_kernel()


@jax.jit
def kernel(x, emb_table, pos_table):
  xT = x.T
  posP = jnp.pad(pos_table, ((0, 0), (0, D)))
  emb2 = emb_table.reshape(500000, 2 * D)
  out3 = _kernel(xT, emb2, posP)
  return out3.transpose(2, 0, 1)


# speedup vs baseline: 2.1721x; 1.1686x over previous
import functools

import jax
import jax.numpy as jnp
from jax import lax
from jax.experimental import pallas as pl
from jax.experimental.pallas import tpu as pltpu
from jax.experimental.pallas import tpu_sc as plsc

B, T, D = 4096, 200, 64
R = B * T
NC, NS = 2, 16
NW = NC * NS
ROWS_PER_W = R // NW
C = 400
NCHUNK = ROWS_PER_W // C
NSTEP = NCHUNK // 2
SUB = 80
LANES = 16


def _make_kernel():
  mesh = plsc.VectorSubcoreMesh(
      core_axis_name="c", subcore_axis_name="s",
      num_cores=NC, num_subcores=NS)

  @functools.partial(
      pl.kernel,
      out_type=jax.ShapeDtypeStruct((R, D), jnp.float32),
      mesh=mesh,
      scratch_types=[
          pltpu.VMEM((T, D), jnp.float32),
          pltpu.VMEM((ROWS_PER_W,), jnp.int32),
          pltpu.VMEM((C, D), jnp.float32),
          pltpu.VMEM((C, D), jnp.float32),
          pltpu.SemaphoreType.DMA,
          pltpu.SemaphoreType.DMA,
          pltpu.SemaphoreType.DMA,
          pltpu.SemaphoreType.DMA,
      ],
      compiler_params=pltpu.CompilerParams(use_tc_tiling_on_sc=False),
  )
  def k(x_hbm, emb_hbm, pos_hbm, out_hbm, pos_v, idx_v,
        rows0, rows1, sem_g0, sem_g1, sem_s0, sem_s1):
    wid = lax.axis_index("s") * NC + lax.axis_index("c")
    w_base = wid * ROWS_PER_W
    rows = (rows0, rows1)
    sem_g = (sem_g0, sem_g1)
    sem_s = (sem_s0, sem_s1)

    pltpu.sync_copy(pos_hbm, pos_v)
    pltpu.sync_copy(x_hbm.at[pl.ds(w_base, ROWS_PER_W)], idx_v)

    def issue_gather(ci, slot):
      for j in range(C // SUB):
        pltpu.async_copy(
            emb_hbm.at[idx_v.at[pl.ds(ci * C + j * SUB, SUB)]],
            rows[slot].at[pl.ds(j * SUB, SUB)],
            sem_g[slot])

    def wait_gather(slot):
      pltpu.make_async_copy(
          emb_hbm.at[idx_v.at[pl.ds(0, C)]], rows[slot], sem_g[slot]).wait()

    def issue_store(ci, slot):
      pltpu.async_copy(
          rows[slot], out_hbm.at[pl.ds(w_base + ci * C, C)], sem_s[slot])

    def wait_store(slot):
      pltpu.make_async_copy(
          rows[slot], out_hbm.at[pl.ds(w_base, C)], sem_s[slot]).wait()

    def add_pos(slot):
      def t_body(t, c2):
        for d in range(D // LANES):
          sl = pl.ds(d * LANES, LANES)
          pv = pos_v[t, sl]
          for s in range(C // T):
            r = s * T + t
            rows[slot][r, sl] = rows[slot][r, sl] + pv
        return c2
      lax.fori_loop(0, T, t_body, 0)

    issue_gather(0, 0)

    def step(i, carry):
      c0 = 2 * i
      wait_gather(0)

      @pl.when(i > 0)
      def _():
        wait_store(1)
      issue_gather(c0 + 1, 1)
      add_pos(0)
      issue_store(c0, 0)

      wait_gather(1)

      @pl.when(i < NSTEP - 1)
      def _():
        wait_store(0)
        issue_gather(c0 + 2, 0)
      add_pos(1)
      issue_store(c0 + 1, 1)
      return carry

    lax.fori_loop(0, NSTEP, step, 0)
    wait_store(0)
    wait_store(1)

  return k


_kernel = _make_kernel()


@jax.jit
def kernel(x, emb_table, pos_table):
  out = _kernel(x.reshape(R), emb_table, pos_table)
  return out.reshape(B, T, D)
